# baseline (speedup 1.0000x reference)
import functools

import jax
import jax.numpy as jnp
from jax import lax
from jax.experimental import pallas as pl
from jax.experimental.pallas import tpu as pltpu
from jax.experimental.pallas import tpu_sc as plsc

NUM_CORES = 2
NUM_SUBCORES = 16
NUM_WORKERS = NUM_CORES * NUM_SUBCORES
CHUNK = 16
NSLOT = 3
GROUP = 16 * CHUNK


def _make_gather(d_model: int, total: int):
    b_per_w = total // NUM_WORKERS
    n_groups = b_per_w // GROUP
    mesh = plsc.VectorSubcoreMesh(
        core_axis_name="c", subcore_axis_name="s", num_cores=NUM_CORES
    )

    @functools.partial(
        pl.kernel,
        out_type=jax.ShapeDtypeStruct((total, d_model), jnp.float32),
        mesh=mesh,
        scratch_types=[
            pltpu.VMEM((b_per_w,), jnp.int32),
            [pltpu.VMEM((CHUNK, d_model), jnp.float32) for _ in range(NSLOT)],
            pltpu.VMEM((CHUNK, d_model), jnp.float32),
            pltpu.VMEM_SHARED((NUM_SUBCORES, NSLOT, CHUNK, d_model), jnp.float32),
            [pltpu.SemaphoreType.DMA for _ in range(NSLOT)],
            [pltpu.SemaphoreType.DMA for _ in range(NSLOT)],
            [pltpu.SemaphoreType.DMA for _ in range(NSLOT)],
            pltpu.SemaphoreType.DMA,
            pltpu.SemaphoreType.DMA,
        ],
    )
    def sc_gather(table_hbm, idx_hbm, out_hbm, idx_v, sbufs, dbuf, shared,
                  gsems, csems, osems, dgsem, dosem):
        wid = lax.axis_index("s") * NUM_CORES + lax.axis_index("c")
        sid = lax.axis_index("s")
        base = wid * b_per_w
        pltpu.sync_copy(idx_hbm.at[pl.ds(base, b_per_w)], idx_v)

        def start_gather(row_off, j):
            idx_slice = idx_v.at[pl.ds(row_off, CHUNK)]
            pltpu.async_copy(table_hbm.at[idx_slice], sbufs[j], gsems[j])

        def wait_gather(j):
            idx_slice = idx_v.at[pl.ds(0, CHUNK)]
            pltpu.make_async_copy(table_hbm.at[idx_slice], sbufs[j],
                                  gsems[j]).wait()

        def start_copy(j):
            pltpu.async_copy(sbufs[j], shared.at[sid, j], csems[j])

        def wait_copy(j):
            pltpu.make_async_copy(sbufs[j], shared.at[sid, j], csems[j]).wait()

        def start_out(row_off, j):
            pltpu.async_copy(shared.at[sid, j],
                             out_hbm.at[pl.ds(base + row_off, CHUNK)], osems[j])

        def wait_out(j):
            pltpu.make_async_copy(shared.at[sid, j],
                                  out_hbm.at[pl.ds(base, CHUNK)], osems[j]).wait()

        def start_dgather(row_off):
            idx_slice = idx_v.at[pl.ds(row_off, CHUNK)]
            pltpu.async_copy(table_hbm.at[idx_slice], dbuf, dgsem)

        def wait_dgather():
            idx_slice = idx_v.at[pl.ds(0, CHUNK)]
            pltpu.make_async_copy(table_hbm.at[idx_slice], dbuf, dgsem).wait()

        def start_dout(row_off):
            pltpu.async_copy(dbuf, out_hbm.at[pl.ds(base + row_off, CHUNK)],
                             dosem)

        def wait_dout():
            pltpu.make_async_copy(dbuf, out_hbm.at[pl.ds(base, CHUNK)],
                                  dosem).wait()

        def group_body(goff, first_group, last_group):
            wait_dgather()
            start_dout(goff + 15 * CHUNK)
            for t in range(5):
                for j in range(NSLOT):
                    i = 3 * t + j
                    wait_gather(j)
                    if not (first_group and t == 0):
                        wait_out(j)
                    start_copy(j)
                    wait_copy(j)
                    start_out(goff + i * CHUNK, j)
                    if t < 4:
                        start_gather(goff + (i + 3) * CHUNK, j)
                    elif not last_group:
                        start_gather(goff + GROUP + j * CHUNK, j)
            wait_dout()
            if not last_group:
                start_dgather(goff + GROUP + 15 * CHUNK)

        for j in range(NSLOT):
            start_gather(j * CHUNK, j)
        start_dgather(15 * CHUNK)

        group_body(0, first_group=True, last_group=False)

        def body(g, carry):
            goff = pl.multiple_of(g * GROUP, GROUP)
            group_body(goff, first_group=False, last_group=False)
            return carry

        lax.fori_loop(1, n_groups - 1, body, 0)

        group_body((n_groups - 1) * GROUP, first_group=False, last_group=True)

        for j in range(NSLOT):
            wait_out(j)

    return sc_gather


def kernel(positions, pe):
    b, s = positions.shape
    n_rows, d_model = pe.shape
    idx = positions.reshape(b * s)
    out = _make_gather(d_model, b * s)(pe, idx)
    return out.reshape(b, s, d_model)


# speedup vs baseline: 1.0224x; 1.0224x over previous
import functools

import jax
import jax.numpy as jnp
from jax import lax
from jax.experimental import pallas as pl
from jax.experimental.pallas import tpu as pltpu
from jax.experimental.pallas import tpu_sc as plsc

NUM_CORES = 2
NUM_SUBCORES = 16
NUM_WORKERS = NUM_CORES * NUM_SUBCORES
CHUNK = 16
SB = 4
SL = 3
UNROLL = 12


def _make_gather(d_model: int, total: int):
    b_per_w = total // NUM_WORKERS
    n_chunks = b_per_w // CHUNK
    assert n_chunks % UNROLL == 4
    mesh = plsc.VectorSubcoreMesh(
        core_axis_name="c", subcore_axis_name="s", num_cores=NUM_CORES
    )

    @functools.partial(
        pl.kernel,
        out_type=jax.ShapeDtypeStruct((total, d_model), jnp.float32),
        mesh=mesh,
        scratch_types=[
            pltpu.VMEM((b_per_w,), jnp.int32),
            [pltpu.VMEM((CHUNK, d_model), jnp.float32) for _ in range(SB)],
            pltpu.VMEM_SHARED((NUM_SUBCORES, SL, CHUNK, d_model), jnp.float32),
            [pltpu.SemaphoreType.DMA for _ in range(SB)],
            [pltpu.SemaphoreType.DMA for _ in range(SB)],
            [pltpu.SemaphoreType.DMA for _ in range(SL)],
        ],
    )
    def sc_gather(table_hbm, idx_hbm, out_hbm, idx_v, sbufs, shared,
                  gsems, csems, osems):
        wid = lax.axis_index("s") * NUM_CORES + lax.axis_index("c")
        sid = lax.axis_index("s")
        base = wid * b_per_w
        pltpu.sync_copy(idx_hbm.at[pl.ds(base, b_per_w)], idx_v)

        def start_gather(row_off, b):
            idx_slice = idx_v.at[pl.ds(row_off, CHUNK)]
            pltpu.async_copy(table_hbm.at[idx_slice], sbufs[b], gsems[b])

        def wait_gather(b):
            idx_slice = idx_v.at[pl.ds(0, CHUNK)]
            pltpu.make_async_copy(table_hbm.at[idx_slice], sbufs[b],
                                  gsems[b]).wait()

        def start_copy(b, j):
            pltpu.async_copy(sbufs[b], shared.at[sid, j], csems[b])

        def wait_copy(b, j):
            pltpu.make_async_copy(sbufs[b], shared.at[sid, j], csems[b]).wait()

        def start_out(row_off, j):
            pltpu.async_copy(shared.at[sid, j],
                             out_hbm.at[pl.ds(base + row_off, CHUNK)], osems[j])

        def wait_out(j):
            pltpu.make_async_copy(shared.at[sid, j],
                                  out_hbm.at[pl.ds(base, CHUNK)], osems[j]).wait()

        def visit(off, c_mod_sb, c_mod_sl, prev_mod_sb, prev_mod_sl,
                  has_prev, has_out_wait, prefetch):
            wait_gather(c_mod_sb)
            if has_prev:
                wait_copy(prev_mod_sb, prev_mod_sl)
                start_out(off - CHUNK, prev_mod_sl)
            if has_out_wait:
                wait_out(c_mod_sl)
            start_copy(c_mod_sb, c_mod_sl)
            if prefetch:
                start_gather(off + 3 * CHUNK, (c_mod_sb + 3) % SB)

        for c in range(3):
            start_gather(c * CHUNK, c % SB)

        visit(0 * CHUNK, 0, 0, 0, 0, False, False, True)
        visit(1 * CHUNK, 1, 1, 0, 0, True, False, True)
        visit(2 * CHUNK, 2, 2, 1, 1, True, False, True)
        visit(3 * CHUNK, 3, 0, 2, 2, True, True, True)

        def body(i, carry):
            off0 = pl.multiple_of((4 + 12 * i) * CHUNK, 4 * CHUNK)
            for k in range(UNROLL):
                cb = k % SB
                cl = (1 + k) % SL
                pb = (k - 1) % SB
                psl = (k) % SL
                visit(off0 + k * CHUNK, cb, cl, pb, psl, True, True, True)
            return carry

        lax.fori_loop(0, (n_chunks - 4 - 3) // UNROLL, body, 0)

        off0 = pl.multiple_of(52 * CHUNK, 4 * CHUNK)
        for k in range(12):
            c = 52 + k
            visit(off0 + k * CHUNK, c % SB, c % SL, (c - 1) % SB, (c - 1) % SL,
                  True, True, c + 3 <= 63)

        wait_copy(63 % SB, 63 % SL)
        start_out(63 * CHUNK, 63 % SL)
        for j in range(SL):
            wait_out(j)

    return sc_gather


def kernel(positions, pe):
    b, s = positions.shape
    n_rows, d_model = pe.shape
    idx = positions.reshape(b * s)
    out = _make_gather(d_model, b * s)(pe, idx)
    return out.reshape(b, s, d_model)


# speedup vs baseline: 1.0291x; 1.0065x over previous
import functools

import jax
import jax.numpy as jnp
from jax import lax
from jax.experimental import pallas as pl
from jax.experimental.pallas import tpu as pltpu
from jax.experimental.pallas import tpu_sc as plsc

NUM_CORES = 2
NUM_SUBCORES = 16
NUM_WORKERS = NUM_CORES * NUM_SUBCORES
CHUNK = 16
NBUF = 3


def _make_gather(d_model: int, total: int):
    b_per_w = total // NUM_WORKERS
    n_chunks = b_per_w // CHUNK
    n_full = n_chunks // NBUF - 1
    tail = n_chunks - NBUF * (n_full + 1)
    mesh = plsc.VectorSubcoreMesh(
        core_axis_name="c", subcore_axis_name="s", num_cores=NUM_CORES
    )

    @functools.partial(
        pl.kernel,
        out_type=jax.ShapeDtypeStruct((total, d_model), jnp.float32),
        mesh=mesh,
        scratch_types=[
            pltpu.VMEM((b_per_w,), jnp.int32),
            [pltpu.VMEM((CHUNK, d_model), jnp.float32) for _ in range(NBUF)],
            pltpu.VMEM_SHARED((NUM_SUBCORES, NBUF, CHUNK, d_model), jnp.float32),
            [pltpu.SemaphoreType.DMA for _ in range(NBUF)],
            [pltpu.SemaphoreType.DMA for _ in range(NBUF)],
            [pltpu.SemaphoreType.DMA for _ in range(NBUF)],
        ],
    )
    def sc_gather(table_hbm, idx_hbm, out_hbm, idx_v, bufs, shared,
                  gsems, csems, osems):
        wid = lax.axis_index("s") * NUM_CORES + lax.axis_index("c")
        sid = lax.axis_index("s")
        base = wid * b_per_w
        pltpu.sync_copy(idx_hbm.at[pl.ds(base, b_per_w)], idx_v)

        def start_gather(chunk_off, k):
            idx_slice = idx_v.at[pl.ds(chunk_off, CHUNK)]
            pltpu.async_copy(table_hbm.at[idx_slice], bufs[k], gsems[k])

        def wait_gather(k):
            idx_slice = idx_v.at[pl.ds(0, CHUNK)]
            pltpu.make_async_copy(table_hbm.at[idx_slice], bufs[k], gsems[k]).wait()

        def start_copy(k):
            pltpu.async_copy(bufs[k], shared.at[sid, k], csems[k])

        def wait_copy(k):
            pltpu.make_async_copy(bufs[k], shared.at[sid, k], csems[k]).wait()

        def start_out(chunk_off, k):
            pltpu.async_copy(shared.at[sid, k],
                             out_hbm.at[pl.ds(base + chunk_off, CHUNK)], osems[k])

        def wait_out(k):
            pltpu.make_async_copy(shared.at[sid, k],
                                  out_hbm.at[pl.ds(base, CHUNK)], osems[k]).wait()

        for k in range(NBUF):
            start_gather(k * CHUNK, k)
        for k in range(NBUF):
            wait_gather(k)
            start_copy(k)
            wait_copy(k)
            start_out(k * CHUNK, k)
            start_gather((k + NBUF) * CHUNK, k)

        def body(p, carry):
            off = pl.multiple_of((p + 1) * (NBUF * CHUNK), NBUF * CHUNK)
            for k in range(NBUF):
                wait_gather(k)
                wait_out(k)
                start_copy(k)
                wait_copy(k)
                start_out(off + k * CHUNK, k)
                start_gather(off + (k + NBUF) * CHUNK, k)
            return carry

        lax.fori_loop(0, n_full - 1, body, 0)

        off = n_full * (NBUF * CHUNK)
        for k in range(NBUF):
            wait_gather(k)
            wait_out(k)
            start_copy(k)
            wait_copy(k)
            start_out(off + k * CHUNK, k)
            t = k
            if t < tail:
                start_gather(off + (k + NBUF) * CHUNK, k)
        off2 = (n_full + 1) * (NBUF * CHUNK)
        for t in range(tail):
            k = t
            wait_gather(k)
            wait_out(k)
            start_copy(k)
            wait_copy(k)
            start_out(off2 + t * CHUNK, k)
        for k in range(NBUF):
            wait_out(k)

    return sc_gather


def kernel(positions, pe):
    b, s = positions.shape
    n_rows, d_model = pe.shape
    idx = positions.reshape(b * s)
    out = _make_gather(d_model, b * s)(pe, idx)
    return out.reshape(b, s, d_model)
